# baseline (speedup 1.0000x reference)
import functools

import jax
import jax.numpy as jnp
from jax import lax
from jax.experimental import pallas as pl
from jax.experimental.pallas import tpu as pltpu
from jax.experimental.pallas import tpu_sc as plsc

_PATCH = 32
_D = 256
_VOCAB = 2 * _PATCH * _PATCH
_B = 32
_N = 8192
_TOK = _B * _N

_NC = 2
_NS = 16
_L = 16
_NW = _NC * _NS
_TPW = _TOK // _NW
_K = 64
_NCH = _TPW // _K
_NBUF = 4
_LOOK = 2


def _prep_body(emb_ref, g_ref, b_ref, ln_ref, sin_ref):
    e = emb_ref[...]
    mu = jnp.mean(e, axis=-1, keepdims=True)
    var = jnp.mean(jnp.square(e - mu), axis=-1, keepdims=True)
    ln_ref[...] = (e - mu) / jnp.sqrt(var + 1e-5) * g_ref[...] + b_ref[...]
    t = lax.broadcasted_iota(jnp.int32, (_PATCH, _D // 2), 0).astype(jnp.float32)
    k = lax.broadcasted_iota(jnp.int32, (_PATCH, _D // 2), 1).astype(jnp.float32)
    freqs = jnp.exp((-jnp.log(10000.0) / (_D // 2)) * k)
    args = t * freqs
    sin_ref[...] = jnp.concatenate([jnp.sin(args), jnp.cos(args)], axis=-1)


def _prep_tables(emb_table, ln_gamma, ln_beta):
    return pl.pallas_call(
        _prep_body,
        out_shape=(
            jax.ShapeDtypeStruct((_VOCAB, _D), jnp.float32),
            jax.ShapeDtypeStruct((_PATCH, _D), jnp.float32),
        ),
    )(emb_table, ln_gamma.reshape(1, _D), ln_beta.reshape(1, _D))


def _sc_body(raw_hbm, ln_hbm, sinflat_hbm, out_hbm,
             raw_v, ids_v, tsoff_v, sin_v, rows_v,
             sem_s, sem_g0, sem_g1, sem_g2, sem_g3,
             sem_o0, sem_o1, sem_o2, sem_o3):
    wid = lax.axis_index("s") * _NC + lax.axis_index("c")
    base = wid * _TPW
    sem_gs = (sem_g0, sem_g1, sem_g2, sem_g3)
    sem_os = (sem_o0, sem_o1, sem_o2, sem_o3)

    pltpu.async_copy(sinflat_hbm, sin_v, sem_s)
    pltpu.sync_copy(raw_hbm.at[pl.ds(base * 4, _TPW * 4)], raw_v)
    pltpu.make_async_copy(sinflat_hbm, sin_v, sem_s).wait()

    def id_body(g, carry):
        lane4 = lax.iota(jnp.int32, _L) * 4 + g * (_L * 4)
        t = plsc.load_gather(raw_v, [lane4])
        x = plsc.load_gather(raw_v, [lane4 + 1])
        y = plsc.load_gather(raw_v, [lane4 + 2])
        p = plsc.load_gather(raw_v, [lane4 + 3])
        out_idx = lax.iota(jnp.int32, _L) + g * _L
        plsc.store_scatter(ids_v, [out_idx], (p + y) * _PATCH + x)
        plsc.store_scatter(tsoff_v, [out_idx], t * _D)
        return carry
    lax.fori_loop(0, _TPW // _L, id_body, 0)

    def row_slice(b):
        return rows_v.at[pl.ds(b * _K, _K)]

    def issue_gather(c, b):
        pltpu.async_copy(ln_hbm.at[ids_v.at[pl.ds(c * _K, _K)]],
                         row_slice(b), sem_gs[b])

    def wait_gather(c, b):
        pltpu.make_async_copy(ln_hbm.at[ids_v.at[pl.ds(c * _K, _K)]],
                              row_slice(b), sem_gs[b]).wait()

    def out_slice(c):
        return out_hbm.at[pl.ds(base + c * _K, _K)]

    for c0 in range(_LOOK):
        issue_gather(c0, c0)

    def outer(cg, carry):
        for b_ in range(_NBUF):
            c = cg * _NBUF + b_
            b = b_
            wait_gather(c, b)

            def add_body(tg, inner):
                tsvec = tsoff_v[pl.ds(c * _K + tg * _L, _L)]
                rowvec = lax.iota(jnp.int32, _L) + (b * _K + tg * _L)
                for col in range(_D):
                    colvec = jnp.full((_L,), col, jnp.int32)
                    svals = plsc.load_gather(sin_v, [tsvec + col])
                    rvals = plsc.load_gather(rows_v, [rowvec, colvec])
                    plsc.store_scatter(rows_v, [rowvec, colvec], rvals + svals)
                return inner
            lax.fori_loop(0, _K // _L, add_body, 0)

            pltpu.async_copy(row_slice(b), out_slice(c), sem_os[b])

            cn = c + _LOOK
            bn = (b_ + _LOOK) % _NBUF

            @pl.when(cn < _NCH)
            def _():
                @pl.when(cn >= _NBUF)
                def _():
                    pltpu.make_async_copy(row_slice(bn), out_slice(cn - _NBUF),
                                          sem_os[bn]).wait()
                issue_gather(cn, bn)
        return carry
    lax.fori_loop(0, _NCH // _NBUF, outer, 0)

    for c in range(_NCH - _NBUF, _NCH):
        b = c % _NBUF
        pltpu.make_async_copy(row_slice(b), out_slice(c), sem_os[b]).wait()


@functools.cache
def _sc_gather():
    return functools.partial(
        pl.kernel,
        out_type=jax.ShapeDtypeStruct((_TOK, _D), jnp.float32),
        mesh=plsc.VectorSubcoreMesh(core_axis_name="c", subcore_axis_name="s",
                                    num_cores=_NC, num_subcores=_NS),
        compiler_params=pltpu.CompilerParams(needs_layout_passes=False),
        scratch_types=[
            pltpu.VMEM((_TPW * 4,), jnp.int32),
            pltpu.VMEM((_TPW,), jnp.int32),
            pltpu.VMEM((_TPW,), jnp.int32),
            pltpu.VMEM((_PATCH * _D,), jnp.float32),
            pltpu.VMEM((_NBUF * _K, _D), jnp.float32),
            pltpu.SemaphoreType.DMA,
            pltpu.SemaphoreType.DMA,
            pltpu.SemaphoreType.DMA,
            pltpu.SemaphoreType.DMA,
            pltpu.SemaphoreType.DMA,
            pltpu.SemaphoreType.DMA,
            pltpu.SemaphoreType.DMA,
            pltpu.SemaphoreType.DMA,
            pltpu.SemaphoreType.DMA,
        ],
    )(_sc_body)


@jax.jit
def kernel(input, emb_table, ln_gamma, ln_beta):
    ln_table, sin_table = _prep_tables(emb_table, ln_gamma, ln_beta)
    raw = input.reshape(_TOK * 4)
    out = _sc_gather()(raw, ln_table, sin_table.reshape(_PATCH * _D))
    return out.reshape(_B, _N, _D)


# speedup vs baseline: 4.7272x; 4.7272x over previous
import functools

import jax
import jax.numpy as jnp
from jax import lax
from jax.experimental import pallas as pl
from jax.experimental.pallas import tpu as pltpu
from jax.experimental.pallas import tpu_sc as plsc

_PATCH = 32
_D = 256
_VOCAB = 2 * _PATCH * _PATCH
_B = 32
_N = 8192
_TOK = _B * _N

_NC = 2
_NS = 16
_L = 16
_NW = _NC * _NS
_TPW = _TOK // _NW
_K = 64
_NCH = _TPW // _K
_NBUF = 2
_LOOK = 2


def _prep_body(emb_ref, g_ref, b_ref, ln_ref, sin_ref):
    e = emb_ref[...]
    mu = jnp.mean(e, axis=-1, keepdims=True)
    var = jnp.mean(jnp.square(e - mu), axis=-1, keepdims=True)
    ln_ref[...] = (e - mu) / jnp.sqrt(var + 1e-5) * g_ref[...] + b_ref[...]
    t = lax.broadcasted_iota(jnp.int32, (_PATCH, _D // 2), 0).astype(jnp.float32)
    k = lax.broadcasted_iota(jnp.int32, (_PATCH, _D // 2), 1).astype(jnp.float32)
    freqs = jnp.exp((-jnp.log(10000.0) / (_D // 2)) * k)
    args = t * freqs
    sin_ref[...] = jnp.concatenate([jnp.sin(args), jnp.cos(args)], axis=-1)


def _prep_tables(emb_table, ln_gamma, ln_beta):
    return pl.pallas_call(
        _prep_body,
        out_shape=(
            jax.ShapeDtypeStruct((_VOCAB, _D), jnp.float32),
            jax.ShapeDtypeStruct((_PATCH, _D), jnp.float32),
        ),
    )(emb_table, ln_gamma.reshape(1, _D), ln_beta.reshape(1, _D))


def _sc_body(raw_hbm, ln_hbm, sinflat_hbm, out_hbm,
             raw_v, ids_v, tsoff_v, sin_v, rows_v, dst_v,
             sem_s, sem_g0, sem_g1, sem_o0, sem_o1):
    wid = lax.axis_index("s") * _NC + lax.axis_index("c")
    base = wid * _TPW
    sem_gs = (sem_g0, sem_g1)
    sem_os = (sem_o0, sem_o1)

    pltpu.async_copy(sinflat_hbm, sin_v, sem_s)
    pltpu.sync_copy(raw_hbm.at[pl.ds(base * 4, _TPW * 4)], raw_v)
    pltpu.make_async_copy(sinflat_hbm, sin_v, sem_s).wait()

    def id_body(g, carry):
        lane4 = lax.iota(jnp.int32, _L) * 4 + g * (_L * 4)
        t = plsc.load_gather(raw_v, [lane4])
        x = plsc.load_gather(raw_v, [lane4 + 1])
        y = plsc.load_gather(raw_v, [lane4 + 2])
        p = plsc.load_gather(raw_v, [lane4 + 3])
        out_idx = lax.iota(jnp.int32, _L) + g * _L
        plsc.store_scatter(ids_v, [out_idx], (p + y) * _PATCH + x)
        plsc.store_scatter(tsoff_v, [out_idx], t * _D)
        return carry
    lax.fori_loop(0, _TPW // _L, id_body, 0)

    def row_slice(b):
        return rows_v.at[pl.ds(b * _K, _K)]

    def dst_slice(b):
        return dst_v.at[pl.ds(b * _K, _K)]

    def issue_gather(c, b):
        pltpu.async_copy(ln_hbm.at[ids_v.at[pl.ds(c * _K, _K)]],
                         row_slice(b), sem_gs[b])

    def wait_gather(c, b):
        pltpu.make_async_copy(ln_hbm.at[ids_v.at[pl.ds(c * _K, _K)]],
                              row_slice(b), sem_gs[b]).wait()

    def out_slice(c):
        return out_hbm.at[pl.ds(base + c * _K, _K)]

    for c0 in range(_LOOK):
        issue_gather(c0, c0 % _NBUF)

    def outer(cg, carry):
        for b in range(_NBUF):
            c = cg * _NBUF + b
            wait_gather(c, b)

            @pl.when(c >= _NBUF)
            def _():
                pltpu.make_async_copy(dst_slice(b), out_slice(c - _NBUF),
                                      sem_os[b]).wait()

            def add_body(tg, inner):
                tsvec = tsoff_v[pl.ds(c * _K + tg * _L, _L)]
                for l in range(_L):
                    sbase = lax.iota(jnp.int32, _L) + tsvec[l]
                    row = b * _K + tg * _L + l
                    for j in range(_D // _L):
                        sl = pl.ds(j * _L, _L)
                        svals = plsc.load_gather(sin_v, [sbase + j * _L])
                        dst_v[row, sl] = rows_v[row, sl] + svals
                return inner
            lax.fori_loop(0, _K // _L, add_body, 0)

            pltpu.async_copy(dst_slice(b), out_slice(c), sem_os[b])

            cn = c + _LOOK

            @pl.when(cn < _NCH)
            def _():
                issue_gather(cn, b)
        return carry
    lax.fori_loop(0, _NCH // _NBUF, outer, 0)

    for c in range(_NCH - _NBUF, _NCH):
        b = c % _NBUF
        pltpu.make_async_copy(dst_slice(b), out_slice(c), sem_os[b]).wait()


@functools.cache
def _sc_gather():
    return functools.partial(
        pl.kernel,
        out_type=jax.ShapeDtypeStruct((_TOK, _D), jnp.float32),
        mesh=plsc.VectorSubcoreMesh(core_axis_name="c", subcore_axis_name="s",
                                    num_cores=_NC, num_subcores=_NS),
        compiler_params=pltpu.CompilerParams(needs_layout_passes=False),
        scratch_types=[
            pltpu.VMEM((_TPW * 4,), jnp.int32),
            pltpu.VMEM((_TPW,), jnp.int32),
            pltpu.VMEM((_TPW,), jnp.int32),
            pltpu.VMEM((_PATCH * _D,), jnp.float32),
            pltpu.VMEM((_NBUF * _K, _D), jnp.float32),
            pltpu.VMEM((_NBUF * _K, _D), jnp.float32),
            pltpu.SemaphoreType.DMA,
            pltpu.SemaphoreType.DMA,
            pltpu.SemaphoreType.DMA,
            pltpu.SemaphoreType.DMA,
            pltpu.SemaphoreType.DMA,
        ],
    )(_sc_body)


@jax.jit
def kernel(input, emb_table, ln_gamma, ln_beta):
    ln_table, sin_table = _prep_tables(emb_table, ln_gamma, ln_beta)
    raw = input.reshape(_TOK * 4)
    out = _sc_gather()(raw, ln_table, sin_table.reshape(_PATCH * _D))
    return out.reshape(_B, _N, _D)
